# baseline (speedup 1.0000x reference)
import functools
import math

import jax
import jax.numpy as jnp
from jax import lax
from jax.experimental import pallas as pl
from jax.experimental.pallas import tpu as pltpu
from jax.experimental.pallas import tpu_sc as plsc

_B = 64
_S = 512
_N = _B * _S
_D4 = 32
_D = 128
_PACK = _D // _D4
_NP = _N // _PACK

_NC = 2
_NS = 16
_NW = _NC * _NS
_B_PER_W = _N // _NW
_CHUNK = 128
_NCHUNK = _B_PER_W // _CHUNK

_RPB = 1024
_SEQ_PER_BLOCK = _RPB * _PACK // _S


def _sc_gather_kernel(table_hbm, idx_hbm, out_hbm, idx_v, rows_v, sem):
    wid = lax.axis_index("s") * _NC + lax.axis_index("c")
    base = wid * _B_PER_W
    pltpu.sync_copy(idx_hbm.at[pl.ds(base, _B_PER_W)], idx_v)
    copies = []
    for j in range(_NCHUNK):
        copies.append(
            pltpu.async_copy(
                table_hbm.at[idx_v.at[pl.ds(j * _CHUNK, _CHUNK)]],
                rows_v.at[pl.ds(j * _CHUNK, _CHUNK)],
                sem,
            )
        )
    for c in copies:
        c.wait()
    pltpu.sync_copy(rows_v, out_hbm.at[pl.ds(base, _B_PER_W)])


def _sc_gather(token_table, idx_flat):
    mesh = plsc.VectorSubcoreMesh(core_axis_name="c", subcore_axis_name="s")
    k = pl.kernel(
        _sc_gather_kernel,
        out_type=jax.ShapeDtypeStruct((_N, _D4), jnp.float32),
        mesh=mesh,
        compiler_params=pltpu.CompilerParams(use_tc_tiling_on_sc=False),
        scratch_types=[
            pltpu.VMEM((_B_PER_W,), jnp.int32),
            pltpu.VMEM((_B_PER_W, _D4), jnp.float32),
            pltpu.SemaphoreType.DMA,
        ],
    )
    return k(token_table, idx_flat)


def _tc_body(g_ref, w_ref, b_ref, p0_ref, p1_ref, p2_ref, p3_ref,
             t0_ref, t1_ref, t2_ref, t3_ref, type_ref, gm_ref, bt_ref,
             out_ref):
    g = g_ref[...]
    h = 0.5 * g * (1.0 + lax.erf(g * (1.0 / math.sqrt(2.0))))
    w = w_ref[...]
    c4 = jnp.concatenate([w, w, w, w], axis=0)
    wrep = jnp.concatenate([c4, c4, c4, c4], axis=1)
    rowq = lax.broadcasted_iota(jnp.int32, (_D, _PACK * _D), 0) // _D4
    colq = lax.broadcasted_iota(jnp.int32, (_D, _PACK * _D), 1) // _D
    w512 = jnp.where(rowq == colq, wrep, 0.0)
    h2 = jnp.dot(h, w512, preferred_element_type=jnp.float32)

    bias = b_ref[...].reshape(1, 1, _D)
    ty0 = type_ref[0, :].reshape(1, _D)
    tyd = (type_ref[1, :] - type_ref[0, :]).reshape(1, _D)
    pos_refs = (p0_ref, p1_ref, p2_ref, p3_ref)
    tt_refs = (t0_ref, t1_ref, t2_ref, t3_ref)
    gm = gm_ref[...].reshape(1, 1, _D)
    bt = bt_ref[...].reshape(1, 1, _D)
    pieces = []
    for k in range(_PACK):
        s = h2[:, k * _D:(k + 1) * _D]
        s = s + tt_refs[k][...] * tyd + ty0
        s = s.reshape(_SEQ_PER_BLOCK, _S // _PACK, _D)
        s = s + pos_refs[k][...][None, :, :] + bias
        mean = jnp.mean(s, axis=-1, keepdims=True)
        d = s - mean
        var = jnp.mean(d * d, axis=-1, keepdims=True)
        s = (d * lax.rsqrt(var + 1e-12)) * gm + bt
        pieces.append(s.reshape(_RPB, _D))
    out_ref[...] = jnp.concatenate(pieces, axis=1)


def _tc_compute(gathered, proj_W, proj_b, pos_k, tt_k, type_table, gamma,
                beta):
    grid = (_NP // _RPB,)
    full = lambda i: (0, 0)
    return pl.pallas_call(
        _tc_body,
        grid=grid,
        in_specs=[
            pl.BlockSpec((_RPB, _D), lambda i: (i, 0)),
            pl.BlockSpec((_D4, _D), full),
            pl.BlockSpec((1, _D), full),
            pl.BlockSpec((_S // _PACK, _D), full),
            pl.BlockSpec((_S // _PACK, _D), full),
            pl.BlockSpec((_S // _PACK, _D), full),
            pl.BlockSpec((_S // _PACK, _D), full),
            pl.BlockSpec((_RPB, 1), lambda i: (i, 0)),
            pl.BlockSpec((_RPB, 1), lambda i: (i, 0)),
            pl.BlockSpec((_RPB, 1), lambda i: (i, 0)),
            pl.BlockSpec((_RPB, 1), lambda i: (i, 0)),
            pl.BlockSpec((2, _D), full),
            pl.BlockSpec((1, _D), full),
            pl.BlockSpec((1, _D), full),
        ],
        out_specs=pl.BlockSpec((_RPB, _PACK * _D), lambda i: (i, 0)),
        out_shape=jax.ShapeDtypeStruct((_NP, _PACK * _D), jnp.float32),
    )(gathered, proj_W, proj_b, *pos_k, *tt_k, type_table, gamma, beta)


def kernel(x, token_type, token_table, proj_W, proj_b, pos_table, type_table,
           gamma, beta):
    idx_flat = x.reshape(_N)
    gathered = _sc_gather(token_table, idx_flat)
    pos = pos_table[:_S]
    pos_k = tuple(pos[k::_PACK] for k in range(_PACK))
    ttf = token_type.reshape(_NP, _PACK).astype(jnp.float32)
    tt_k = tuple(ttf[:, k:k + 1] for k in range(_PACK))
    out = _tc_compute(
        gathered.reshape(_NP, _D),
        proj_W,
        proj_b.reshape(1, _D),
        pos_k,
        tt_k,
        type_table,
        gamma.reshape(1, _D),
        beta.reshape(1, _D),
    )
    return out.reshape(_B, _S, _D)
